# Initial kernel scaffold; baseline (speedup 1.0000x reference)
import jax
import jax.numpy as jnp
from jax.experimental import pallas as pl


def kernel(logits, metro_idx, s_weight, b_weight):
    raise NotImplementedError("write your pallas kernel here")


# speedup vs baseline: 7.8118x; 7.8118x over previous
import functools

import jax
import jax.numpy as jnp
from jax import lax
from jax.experimental import pallas as pl
from jax.experimental.pallas import tpu as pltpu
from jax.experimental.pallas import tpu_sc as plsc

_ALPHA = 0.2
_BETA = 0.2
_N_METROS = 1000
_BATCH = 16384

_NC = 2
_NS = 16
_NW = _NC * _NS
_BPW = _BATCH // _NW
_L = 16

_mesh = plsc.VectorSubcoreMesh(core_axis_name="c", subcore_axis_name="s")


@functools.partial(
    pl.kernel,
    out_type=jax.ShapeDtypeStruct((_BATCH,), jnp.float32),
    mesh=_mesh,
    compiler_params=pltpu.CompilerParams(needs_layout_passes=False),
    scratch_types=[
        pltpu.VMEM((_BPW,), jnp.int32),
        pltpu.VMEM((_BPW,), jnp.float32),
        pltpu.VMEM((_BPW,), jnp.float32),
        pltpu.VMEM((_N_METROS,), jnp.float32),
        pltpu.VMEM((_N_METROS,), jnp.float32),
    ],
)
def _metro_affine(logits_hbm, idx_hbm, s_hbm, b_hbm, out_hbm,
                  idx_v, lg_v, out_v, s_v, b_v):
    wid = lax.axis_index("s") * _NC + lax.axis_index("c")
    base = wid * _BPW
    pltpu.sync_copy(idx_hbm.at[pl.ds(base, _BPW)], idx_v)
    pltpu.sync_copy(logits_hbm.at[pl.ds(base, _BPW)], lg_v)
    pltpu.sync_copy(s_hbm, s_v)
    pltpu.sync_copy(b_hbm, b_v)
    for i in range(_BPW // _L):
        sl = pl.ds(i * _L, _L)
        idx = idx_v[sl]
        s_raw = plsc.load_gather(s_v, [idx])
        b_raw = plsc.load_gather(b_v, [idx])
        lg = lg_v[sl]
        tanh_s = 1.0 - 2.0 / (jnp.exp(s_raw * 2.0) + 1.0)
        tanh_b = 1.0 - 2.0 / (jnp.exp(b_raw * 2.0) + 1.0)
        out_v[sl] = lg * (1.0 + _ALPHA * tanh_s) + _BETA * tanh_b
    pltpu.sync_copy(out_v, out_hbm.at[pl.ds(base, _BPW)])


def kernel(logits, metro_idx, s_weight, b_weight):
    idx = metro_idx.astype(jnp.int32)
    return _metro_affine(logits, idx,
                         s_weight.reshape(_N_METROS),
                         b_weight.reshape(_N_METROS))


# speedup vs baseline: 8.0858x; 1.0351x over previous
import functools

import jax
import jax.numpy as jnp
from jax import lax
from jax.experimental import pallas as pl
from jax.experimental.pallas import tpu as pltpu
from jax.experimental.pallas import tpu_sc as plsc

_ALPHA = 0.2
_BETA = 0.2
_N_METROS = 1000
_BATCH = 16384

_NC = 2
_NS = 16
_NW = _NC * _NS
_BPW = _BATCH // _NW
_L = 16

_mesh = plsc.VectorSubcoreMesh(core_axis_name="c", subcore_axis_name="s")


@functools.partial(
    pl.kernel,
    out_type=jax.ShapeDtypeStruct((_BATCH,), jnp.float32),
    mesh=_mesh,
    compiler_params=pltpu.CompilerParams(needs_layout_passes=False),
    scratch_types=[
        pltpu.VMEM((_BPW,), jnp.int32),
        pltpu.VMEM((_BPW,), jnp.float32),
        pltpu.VMEM((_BPW,), jnp.float32),
        pltpu.VMEM((_N_METROS,), jnp.float32),
        pltpu.VMEM((_N_METROS,), jnp.float32),
        pltpu.SemaphoreType.DMA,
    ],
)
def _metro_affine(logits_hbm, idx_hbm, s_hbm, b_hbm, out_hbm,
                  idx_v, lg_v, out_v, s_v, b_v, sem):
    wid = lax.axis_index("s") * _NC + lax.axis_index("c")
    base = wid * _BPW
    cp_i = pltpu.async_copy(idx_hbm.at[pl.ds(base, _BPW)], idx_v, sem)
    cp_l = pltpu.async_copy(logits_hbm.at[pl.ds(base, _BPW)], lg_v, sem)
    cp_s = pltpu.async_copy(s_hbm, s_v, sem)
    cp_b = pltpu.async_copy(b_hbm, b_v, sem)
    cp_i.wait()
    cp_l.wait()
    cp_s.wait()
    cp_b.wait()
    for i in range(_BPW // _L):
        sl = pl.ds(i * _L, _L)
        idx = idx_v[sl]
        s_raw = plsc.load_gather(s_v, [idx])
        b_raw = plsc.load_gather(b_v, [idx])
        lg = lg_v[sl]
        tanh_s = 1.0 - 2.0 / (jnp.exp(s_raw * 2.0) + 1.0)
        tanh_b = 1.0 - 2.0 / (jnp.exp(b_raw * 2.0) + 1.0)
        out_v[sl] = lg * (1.0 + _ALPHA * tanh_s) + _BETA * tanh_b
    pltpu.sync_copy(out_v, out_hbm.at[pl.ds(base, _BPW)])


def kernel(logits, metro_idx, s_weight, b_weight):
    idx = metro_idx.astype(jnp.int32)
    return _metro_affine(logits, idx,
                         s_weight.reshape(_N_METROS),
                         b_weight.reshape(_N_METROS))
